# baseline (speedup 1.0000x reference)
import functools

import jax
import jax.numpy as jnp
from jax import lax
from jax.experimental import pallas as pl
from jax.experimental.pallas import tpu as pltpu
from jax.experimental.pallas import tpu_sc as plsc

NC = 2
NS = 16
NW = NC * NS
L = 16
S = 5
C = 4
H = 512
W = 1024
HW = H * W
RPT = H // NW
PPT = RPT * W
VECS = PPT // L
CV = W // L
NROW = 4 + 4 * C + C
PBLK = 32 * L


def _mesh():
    return plsc.VectorSubcoreMesh(
        core_axis_name="c", subcore_axis_name="s", num_cores=NC, num_subcores=NS
    )


def _wid():
    return lax.axis_index("s") * NC + lax.axis_index("c")


def _vsqrt(x):
    xi = lax.bitcast_convert_type(x, jnp.int32)
    yi = jnp.int32(0x5F3759DF) - (xi >> 1)
    r = lax.bitcast_convert_type(yi, jnp.float32)
    x2 = 0.5 * x
    r = r * (1.5 - x2 * r * r)
    r = r * (1.5 - x2 * r * r)
    r = r * (1.5 - x2 * r * r)
    return jnp.where(x > 0.0, x * r, 0.0)


def _hsum(v):
    idx = lax.iota(jnp.int32, L)
    for sh in (8, 4, 2, 1):
        v = v + v.at[idx ^ sh].get(mode="promise_in_bounds")
    return v


def _vec(i):
    return i >> 6, pl.multiple_of((i & (CV - 1)) << 4, L)


def _pass1_body(pred_hbm, lab_hbm, dv_hbm, dd_hbm, out_hbm,
                lab_v, pred_v, part_v, sem):
    del dv_hbm, dd_hbm
    wid = _wid()
    r0 = wid * RPT
    cps = [pltpu.async_copy(lab_hbm.at[pl.ds(r0, RPT), :], lab_v, sem)]
    for c in range(C):
        cps.append(
            pltpu.async_copy(pred_hbm.at[c, pl.ds(r0, RPT), :], pred_v.at[c], sem)
        )
    for cp in cps:
        cp.wait()

    zero = jnp.zeros((L,), jnp.float32)

    def body(i, acc):
        cnt, sums, tot = acc
        r, cc = _vec(i)
        lab16 = lab_v[r, pl.ds(cc, L)]
        p = [pred_v[c, r, pl.ds(cc, L)] for c in range(C)]
        cnt = list(cnt)
        sums = [list(row) for row in sums]
        tot = list(tot)
        for s in range(S - 1):
            m = lab16 == s
            cnt[s] = cnt[s] + jnp.where(m, 1.0, 0.0)
            for c in range(C):
                sums[s][c] = sums[s][c] + jnp.where(m, p[c], 0.0)
        for c in range(C):
            tot[c] = tot[c] + p[c]
        return (
            tuple(cnt),
            tuple(tuple(row) for row in sums),
            tuple(tot),
        )

    cnt0 = tuple(zero for _ in range(S - 1))
    sums0 = tuple(tuple(zero for _ in range(C)) for _ in range(S - 1))
    tot0 = tuple(zero for _ in range(C))
    cnt, sums, tot = plsc.parallel_loop(
        0, VECS, carry=(cnt0, sums0, tot0), unroll=4
    )(body)

    for s in range(S - 1):
        part_v[pl.ds(s * L, L)] = cnt[s]
        for c in range(C):
            part_v[pl.ds((4 + s * C + c) * L, L)] = sums[s][c]
    for c in range(C):
        part_v[pl.ds((4 + 4 * C + c) * L, L)] = tot[c]
    pltpu.sync_copy(part_v, out_hbm.at[pl.ds(wid * PBLK, PBLK)])


def _pass2_body(pred_hbm, lab_hbm, p1_hbm, dv_hbm, dd_hbm, out_hbm,
                lab_v, pred_v, p1_v, dv_v, dd_v, outv, sem):
    wid = _wid()
    r0 = wid * RPT
    cps = [pltpu.async_copy(lab_hbm.at[pl.ds(r0, RPT), :], lab_v, sem)]
    for c in range(C):
        cps.append(
            pltpu.async_copy(pred_hbm.at[c, pl.ds(r0, RPT), :], pred_v.at[c], sem)
        )
    cps.append(pltpu.async_copy(p1_hbm, p1_v, sem))
    cps.append(pltpu.async_copy(dv_hbm, dv_v, sem))
    cps.append(pltpu.async_copy(dd_hbm, dd_v, sem))
    for cp in cps:
        cp.wait()
    dv = dv_v[...]
    dd = dd_v[...]

    def comb(t, acc):
        return tuple(
            acc[j] + p1_v[pl.ds(t * PBLK + j * L, L)] for j in range(NROW)
        )

    tot = lax.fori_loop(
        1, NW, comb, tuple(p1_v[pl.ds(j * L, L)] for j in range(NROW))
    )

    one = jnp.ones((L,), jnp.float32)
    zero = jnp.zeros((L,), jnp.float32)

    cnt = [_hsum(tot[s]) for s in range(S - 1)]
    cnt4 = jnp.full((L,), float(HW), jnp.float32)
    for s in range(S - 1):
        cnt4 = cnt4 - cnt[s]
    cnt.append(cnt4)
    present = [cnt[s] > 0.0 for s in range(S)]
    cnt_safe = [jnp.where(present[s], cnt[s], one) for s in range(S)]
    kvec = zero
    for s in range(S):
        kvec = kvec + jnp.where(present[s], one, zero)
    sums = [[_hsum(tot[4 + s * C + c]) for c in range(C)] for s in range(S - 1)]
    last = []
    for c in range(C):
        sc = _hsum(tot[4 + 4 * C + c])
        for s in range(S - 1):
            sc = sc - sums[s][c]
        last.append(sc)
    sums.append(last)
    mu = [
        [sums[s][c] / cnt_safe[s] for c in range(C)]
        for s in range(S)
    ]

    def body(i, acc):
        seg, totd = acc
        r, cc = _vec(i)
        lab16 = lab_v[r, pl.ds(cc, L)]
        p = [pred_v[c, r, pl.ds(cc, L)] for c in range(C)]
        masks = [lab16 == s for s in range(S - 1)]
        sq = zero
        for c in range(C):
            mc = mu[S - 1][c]
            for s in range(S - 2, -1, -1):
                mc = jnp.where(masks[s], mu[s][c], mc)
            dis = mc - p[c]
            sq = sq + dis * dis
        nrm = _vsqrt(sq)
        h = jnp.maximum(nrm - dv, 0.0)
        d = h * h
        seg = tuple(
            seg[s] + jnp.where(masks[s], d, zero) for s in range(S - 1)
        )
        return seg, totd + d

    seg, totd = plsc.parallel_loop(
        0, VECS, carry=(tuple(zero for _ in range(S - 1)), zero), unroll=4
    )(body)
    seg = list(seg)
    seg4 = totd
    for s in range(S - 1):
        seg4 = seg4 - seg[s]
    seg.append(seg4)

    part = zero
    for s in range(S):
        part = part + jnp.where(
            present[s], seg[s] / (cnt_safe[s] * kvec), zero
        )

    acc = zero
    for a in range(S):
        for b in range(a + 1, S):
            sq2 = zero
            for c in range(C):
                df = mu[a][c] - mu[b][c]
                sq2 = sq2 + df * df
            dist = _vsqrt(sq2)
            hg = jnp.maximum(dd - dist, 0.0)
            pm = jnp.where(present[a], one, zero) * jnp.where(
                present[b], one, zero
            )
            acc = acc + 2.0 * pm * hg * hg
    l_dist = acc / (kvec * (kvec - one))
    part = part + l_dist * (1.0 / (NW * L))

    outv[...] = part
    pltpu.sync_copy(outv, out_hbm.at[pl.ds(wid * L, L)])


@functools.lru_cache(maxsize=1)
def _build():
    mesh = _mesh()
    params = pltpu.CompilerParams(
        use_tc_tiling_on_sc=True, skip_device_barrier=True
    )
    p1 = pl.kernel(
        _pass1_body,
        out_type=jax.ShapeDtypeStruct((NW * PBLK,), jnp.float32),
        mesh=mesh,
        compiler_params=params,
        scratch_types=[
            pltpu.VMEM((RPT, W), jnp.int32),
            pltpu.VMEM((C, RPT, W), jnp.float32),
            pltpu.VMEM((PBLK,), jnp.float32),
            pltpu.SemaphoreType.DMA,
        ],
    )
    p2 = pl.kernel(
        _pass2_body,
        out_type=jax.ShapeDtypeStruct((NW * L,), jnp.float32),
        mesh=mesh,
        compiler_params=params,
        scratch_types=[
            pltpu.VMEM((RPT, W), jnp.int32),
            pltpu.VMEM((C, RPT, W), jnp.float32),
            pltpu.VMEM((NW * PBLK,), jnp.float32),
            pltpu.VMEM((L,), jnp.float32),
            pltpu.VMEM((L,), jnp.float32),
            pltpu.VMEM((L,), jnp.float32),
            pltpu.SemaphoreType.DMA,
        ],
    )
    return p1, p2


def kernel(prediction, correct_label, delta_v, delta_d):
    pass1, pass2 = _build()
    pred = prediction.reshape(C, H, W)
    lab = correct_label.reshape(H, W).astype(jnp.int32)
    dv = jnp.full((L,), delta_v, jnp.float32)
    dd = jnp.full((L,), delta_d, jnp.float32)
    p1 = pass1(pred, lab, dv, dd)
    parts = pass2(pred, lab, p1, dv, dd)
    return jnp.sum(parts)


# speedup vs baseline: 1.0117x; 1.0117x over previous
import functools

import jax
import jax.numpy as jnp
from jax import lax
from jax.experimental import pallas as pl
from jax.experimental.pallas import tpu as pltpu
from jax.experimental.pallas import tpu_sc as plsc

NC = 2
NS = 16
NW = NC * NS
L = 16
S = 5
C = 4
H = 512
W = 1024
HW = H * W
RPT = H // NW
PPT = RPT * W
VECS = PPT // L
CV = W // L
NROW = 4 + 4 * C + C
PBLK = 32 * L


def _mesh():
    return plsc.VectorSubcoreMesh(
        core_axis_name="c", subcore_axis_name="s", num_cores=NC, num_subcores=NS
    )


def _wid():
    return lax.axis_index("s") * NC + lax.axis_index("c")


def _vsqrt(x):
    xi = lax.bitcast_convert_type(x, jnp.int32)
    yi = jnp.int32(0x5F3759DF) - (xi >> 1)
    r = lax.bitcast_convert_type(yi, jnp.float32)
    x2 = 0.5 * x
    r = r * (1.5 - x2 * r * r)
    r = r * (1.5 - x2 * r * r)
    return jnp.where(x > 0.0, x * r, 0.0)


def _hsum(v):
    idx = lax.iota(jnp.int32, L)
    for sh in (8, 4, 2, 1):
        v = v + v.at[idx ^ sh].get(mode="promise_in_bounds")
    return v


def _vec(i):
    return i >> 6, pl.multiple_of((i & (CV - 1)) << 4, L)


def _pass1_body(pred_hbm, lab_hbm, dv_hbm, dd_hbm, out_hbm,
                lab_v, pred_v, part_v, sem):
    del dv_hbm, dd_hbm
    wid = _wid()
    r0 = wid * RPT
    cps = [pltpu.async_copy(lab_hbm.at[pl.ds(r0, RPT), :], lab_v, sem)]
    for c in range(C):
        cps.append(
            pltpu.async_copy(pred_hbm.at[c, pl.ds(r0, RPT), :], pred_v.at[c], sem)
        )
    for cp in cps:
        cp.wait()

    zero = jnp.zeros((L,), jnp.float32)

    def body(i, acc):
        cnt, sums, tot = acc
        r, cc = _vec(i)
        lab16 = lab_v[r, pl.ds(cc, L)]
        p = [pred_v[c, r, pl.ds(cc, L)] for c in range(C)]
        cnt = list(cnt)
        sums = [list(row) for row in sums]
        tot = list(tot)
        for s in range(S - 1):
            m = lab16 == s
            cnt[s] = cnt[s] + jnp.where(m, 1.0, 0.0)
            for c in range(C):
                sums[s][c] = sums[s][c] + jnp.where(m, p[c], 0.0)
        for c in range(C):
            tot[c] = tot[c] + p[c]
        return (
            tuple(cnt),
            tuple(tuple(row) for row in sums),
            tuple(tot),
        )

    cnt0 = tuple(zero for _ in range(S - 1))
    sums0 = tuple(tuple(zero for _ in range(C)) for _ in range(S - 1))
    tot0 = tuple(zero for _ in range(C))
    cnt, sums, tot = plsc.parallel_loop(
        0, VECS, carry=(cnt0, sums0, tot0), unroll=4
    )(body)

    for s in range(S - 1):
        part_v[pl.ds(s * L, L)] = cnt[s]
        for c in range(C):
            part_v[pl.ds((4 + s * C + c) * L, L)] = sums[s][c]
    for c in range(C):
        part_v[pl.ds((4 + 4 * C + c) * L, L)] = tot[c]
    pltpu.sync_copy(part_v, out_hbm.at[pl.ds(wid * PBLK, PBLK)])


def _pass2_body(pred_hbm, lab_hbm, p1_hbm, dv_hbm, dd_hbm, out_hbm,
                lab_v, pred_v, p1_v, dv_v, dd_v, outv, sem):
    wid = _wid()
    r0 = wid * RPT
    cps = [pltpu.async_copy(lab_hbm.at[pl.ds(r0, RPT), :], lab_v, sem)]
    for c in range(C):
        cps.append(
            pltpu.async_copy(pred_hbm.at[c, pl.ds(r0, RPT), :], pred_v.at[c], sem)
        )
    cps.append(pltpu.async_copy(p1_hbm, p1_v, sem))
    cps.append(pltpu.async_copy(dv_hbm, dv_v, sem))
    cps.append(pltpu.async_copy(dd_hbm, dd_v, sem))
    for cp in cps:
        cp.wait()
    dv = dv_v[...]
    dd = dd_v[...]

    def comb(t, acc):
        return tuple(
            acc[j] + p1_v[pl.ds(t * PBLK + j * L, L)] for j in range(NROW)
        )

    tot = lax.fori_loop(
        1, NW, comb, tuple(p1_v[pl.ds(j * L, L)] for j in range(NROW))
    )

    one = jnp.ones((L,), jnp.float32)
    zero = jnp.zeros((L,), jnp.float32)

    cnt = [_hsum(tot[s]) for s in range(S - 1)]
    cnt4 = jnp.full((L,), float(HW), jnp.float32)
    for s in range(S - 1):
        cnt4 = cnt4 - cnt[s]
    cnt.append(cnt4)
    present = [cnt[s] > 0.0 for s in range(S)]
    cnt_safe = [jnp.where(present[s], cnt[s], one) for s in range(S)]
    kvec = zero
    for s in range(S):
        kvec = kvec + jnp.where(present[s], one, zero)
    sums = [[_hsum(tot[4 + s * C + c]) for c in range(C)] for s in range(S - 1)]
    last = []
    for c in range(C):
        sc = _hsum(tot[4 + 4 * C + c])
        for s in range(S - 1):
            sc = sc - sums[s][c]
        last.append(sc)
    sums.append(last)
    mu = [
        [sums[s][c] / cnt_safe[s] for c in range(C)]
        for s in range(S)
    ]

    def body(i, acc):
        seg, totd = acc
        r, cc = _vec(i)
        lab16 = lab_v[r, pl.ds(cc, L)]
        p = [pred_v[c, r, pl.ds(cc, L)] for c in range(C)]
        masks = [lab16 == s for s in range(S - 1)]
        sq = zero
        for c in range(C):
            mc = mu[S - 1][c]
            for s in range(S - 2, -1, -1):
                mc = jnp.where(masks[s], mu[s][c], mc)
            dis = mc - p[c]
            sq = sq + dis * dis
        nrm = _vsqrt(sq)
        h = jnp.maximum(nrm - dv, 0.0)
        d = h * h
        seg = tuple(
            seg[s] + jnp.where(masks[s], d, zero) for s in range(S - 1)
        )
        return seg, totd + d

    seg, totd = plsc.parallel_loop(
        0, VECS, carry=(tuple(zero for _ in range(S - 1)), zero), unroll=4
    )(body)
    seg = list(seg)
    seg4 = totd
    for s in range(S - 1):
        seg4 = seg4 - seg[s]
    seg.append(seg4)

    part = zero
    for s in range(S):
        part = part + jnp.where(
            present[s], seg[s] / (cnt_safe[s] * kvec), zero
        )

    acc = zero
    for a in range(S):
        for b in range(a + 1, S):
            sq2 = zero
            for c in range(C):
                df = mu[a][c] - mu[b][c]
                sq2 = sq2 + df * df
            dist = _vsqrt(sq2)
            hg = jnp.maximum(dd - dist, 0.0)
            pm = jnp.where(present[a], one, zero) * jnp.where(
                present[b], one, zero
            )
            acc = acc + 2.0 * pm * hg * hg
    l_dist = acc / (kvec * (kvec - one))
    part = part + l_dist * (1.0 / (NW * L))

    outv[...] = part
    pltpu.sync_copy(outv, out_hbm.at[pl.ds(wid * L, L)])


@functools.lru_cache(maxsize=1)
def _build():
    mesh = _mesh()
    params = pltpu.CompilerParams(use_tc_tiling_on_sc=True)
    p1 = pl.kernel(
        _pass1_body,
        out_type=jax.ShapeDtypeStruct((NW * PBLK,), jnp.float32),
        mesh=mesh,
        compiler_params=params,
        scratch_types=[
            pltpu.VMEM((RPT, W), jnp.int32),
            pltpu.VMEM((C, RPT, W), jnp.float32),
            pltpu.VMEM((PBLK,), jnp.float32),
            pltpu.SemaphoreType.DMA,
        ],
    )
    p2 = pl.kernel(
        _pass2_body,
        out_type=jax.ShapeDtypeStruct((NW * L,), jnp.float32),
        mesh=mesh,
        compiler_params=params,
        scratch_types=[
            pltpu.VMEM((RPT, W), jnp.int32),
            pltpu.VMEM((C, RPT, W), jnp.float32),
            pltpu.VMEM((NW * PBLK,), jnp.float32),
            pltpu.VMEM((L,), jnp.float32),
            pltpu.VMEM((L,), jnp.float32),
            pltpu.VMEM((L,), jnp.float32),
            pltpu.SemaphoreType.DMA,
        ],
    )
    return p1, p2


def kernel(prediction, correct_label, delta_v, delta_d):
    pass1, pass2 = _build()
    pred = prediction.reshape(C, H, W)
    lab = correct_label.reshape(H, W).astype(jnp.int32)
    dv = jnp.full((L,), delta_v, jnp.float32)
    dd = jnp.full((L,), delta_d, jnp.float32)
    p1 = pass1(pred, lab, dv, dd)
    parts = pass2(pred, lab, p1, dv, dd)
    return jnp.sum(parts)


# speedup vs baseline: 1.1095x; 1.0967x over previous
import functools

import jax
import jax.numpy as jnp
from jax import lax
from jax.experimental import pallas as pl
from jax.experimental.pallas import tpu as pltpu
from jax.experimental.pallas import tpu_sc as plsc

NC = 2
NS = 16
NW = NC * NS
L = 16
S = 5
C = 4
H = 512
W = 1024
HW = H * W
RPT = H // NW
PPT = RPT * W
VECS = PPT // L
CV = W // L
H_SC = H // 2
RPT2 = H_SC // NW
VECS2 = RPT2 * CV
NROW = 4 + 4 * C + C
PBLK = 32 * L


def _mesh():
    return plsc.VectorSubcoreMesh(
        core_axis_name="c", subcore_axis_name="s", num_cores=NC, num_subcores=NS
    )


def _wid():
    return lax.axis_index("s") * NC + lax.axis_index("c")


def _vsqrt(x):
    xi = lax.bitcast_convert_type(x, jnp.int32)
    yi = jnp.int32(0x5F3759DF) - (xi >> 1)
    r = lax.bitcast_convert_type(yi, jnp.float32)
    x2 = 0.5 * x
    r = r * (1.5 - x2 * r * r)
    r = r * (1.5 - x2 * r * r)
    return jnp.where(x > 0.0, x * r, 0.0)


def _hsum(v):
    idx = lax.iota(jnp.int32, L)
    for sh in (8, 4, 2, 1):
        v = v + v.at[idx ^ sh].get(mode="promise_in_bounds")
    return v


def _vec(i):
    return i >> 6, pl.multiple_of((i & (CV - 1)) << 4, L)


def _pass1_body(pred_hbm, lab_hbm, dv_hbm, dd_hbm, out_hbm,
                lab_v, pred_v, part_v, sem):
    del dv_hbm, dd_hbm
    wid = _wid()
    r0 = wid * RPT
    cps = [pltpu.async_copy(lab_hbm.at[pl.ds(r0, RPT), :], lab_v, sem)]
    for c in range(C):
        cps.append(
            pltpu.async_copy(pred_hbm.at[c, pl.ds(r0, RPT), :], pred_v.at[c], sem)
        )
    for cp in cps:
        cp.wait()

    zero = jnp.zeros((L,), jnp.float32)

    def body(i, acc):
        cnt, sums, tot = acc
        r, cc = _vec(i)
        lab16 = lab_v[r, pl.ds(cc, L)]
        p = [pred_v[c, r, pl.ds(cc, L)] for c in range(C)]
        cnt = list(cnt)
        sums = [list(row) for row in sums]
        tot = list(tot)
        for s in range(S - 1):
            m = lab16 == s
            cnt[s] = cnt[s] + jnp.where(m, 1.0, 0.0)
            for c in range(C):
                sums[s][c] = sums[s][c] + jnp.where(m, p[c], 0.0)
        for c in range(C):
            tot[c] = tot[c] + p[c]
        return (
            tuple(cnt),
            tuple(tuple(row) for row in sums),
            tuple(tot),
        )

    cnt0 = tuple(zero for _ in range(S - 1))
    sums0 = tuple(tuple(zero for _ in range(C)) for _ in range(S - 1))
    tot0 = tuple(zero for _ in range(C))
    cnt, sums, tot = plsc.parallel_loop(
        0, VECS, carry=(cnt0, sums0, tot0), unroll=4
    )(body)

    for s in range(S - 1):
        part_v[pl.ds(s * L, L)] = cnt[s]
        for c in range(C):
            part_v[pl.ds((4 + s * C + c) * L, L)] = sums[s][c]
    for c in range(C):
        part_v[pl.ds((4 + 4 * C + c) * L, L)] = tot[c]
    pltpu.sync_copy(part_v, out_hbm.at[pl.ds(wid * PBLK, PBLK)])


def _pass2_body(pred_hbm, lab_hbm, p1_hbm, dv_hbm, dd_hbm, out_hbm,
                lab_v, pred_v, p1_v, dv_v, dd_v, outv, sem):
    wid = _wid()
    r0 = wid * RPT2
    cps = [pltpu.async_copy(lab_hbm.at[pl.ds(r0, RPT2), :], lab_v, sem)]
    for c in range(C):
        cps.append(
            pltpu.async_copy(pred_hbm.at[c, pl.ds(r0, RPT2), :], pred_v.at[c], sem)
        )
    cps.append(pltpu.async_copy(p1_hbm, p1_v, sem))
    cps.append(pltpu.async_copy(dv_hbm, dv_v, sem))
    cps.append(pltpu.async_copy(dd_hbm, dd_v, sem))
    for cp in cps:
        cp.wait()
    dv = dv_v[...]
    dd = dd_v[...]

    def comb(t, acc):
        return tuple(
            acc[j] + p1_v[pl.ds(t * PBLK + j * L, L)] for j in range(NROW)
        )

    tot = lax.fori_loop(
        1, NW, comb, tuple(p1_v[pl.ds(j * L, L)] for j in range(NROW))
    )

    one = jnp.ones((L,), jnp.float32)
    zero = jnp.zeros((L,), jnp.float32)

    cnt = [_hsum(tot[s]) for s in range(S - 1)]
    cnt4 = jnp.full((L,), float(HW), jnp.float32)
    for s in range(S - 1):
        cnt4 = cnt4 - cnt[s]
    cnt.append(cnt4)
    present = [cnt[s] > 0.0 for s in range(S)]
    cnt_safe = [jnp.where(present[s], cnt[s], one) for s in range(S)]
    kvec = zero
    for s in range(S):
        kvec = kvec + jnp.where(present[s], one, zero)
    sums = [[_hsum(tot[4 + s * C + c]) for c in range(C)] for s in range(S - 1)]
    last = []
    for c in range(C):
        sc = _hsum(tot[4 + 4 * C + c])
        for s in range(S - 1):
            sc = sc - sums[s][c]
        last.append(sc)
    sums.append(last)
    mu = [
        [sums[s][c] / cnt_safe[s] for c in range(C)]
        for s in range(S)
    ]

    def body(i, acc):
        seg, totd = acc
        r, cc = _vec(i)
        lab16 = lab_v[r, pl.ds(cc, L)]
        p = [pred_v[c, r, pl.ds(cc, L)] for c in range(C)]
        masks = [lab16 == s for s in range(S - 1)]
        sq = zero
        for c in range(C):
            mc = mu[S - 1][c]
            for s in range(S - 2, -1, -1):
                mc = jnp.where(masks[s], mu[s][c], mc)
            dis = mc - p[c]
            sq = sq + dis * dis
        nrm = _vsqrt(sq)
        h = jnp.maximum(nrm - dv, 0.0)
        d = h * h
        seg = tuple(
            seg[s] + jnp.where(masks[s], d, zero) for s in range(S - 1)
        )
        return seg, totd + d

    seg, totd = plsc.parallel_loop(
        0, VECS2, carry=(tuple(zero for _ in range(S - 1)), zero), unroll=4
    )(body)
    seg = list(seg)
    seg4 = totd
    for s in range(S - 1):
        seg4 = seg4 - seg[s]
    seg.append(seg4)

    part = zero
    for s in range(S):
        part = part + jnp.where(
            present[s], seg[s] / (cnt_safe[s] * kvec), zero
        )

    acc = zero
    for a in range(S):
        for b in range(a + 1, S):
            sq2 = zero
            for c in range(C):
                df = mu[a][c] - mu[b][c]
                sq2 = sq2 + df * df
            dist = _vsqrt(sq2)
            hg = jnp.maximum(dd - dist, 0.0)
            pm = jnp.where(present[a], one, zero) * jnp.where(
                present[b], one, zero
            )
            acc = acc + 2.0 * pm * hg * hg
    l_dist = acc / (kvec * (kvec - one))
    part = part + l_dist * (1.0 / (NW * L))

    outv[...] = part
    pltpu.sync_copy(outv, out_hbm.at[pl.ds(wid * L, L)])


TC_ROWS = H - H_SC
TC_STEPS = 4
TC_BLK = TC_ROWS // TC_STEPS


def _tc_body(tot_ref, dv_ref, pred_ref, lab_ref, out_ref):
    tot = [tot_ref[0, j] for j in range(NROW)]
    cnt = [tot[s] for s in range(S - 1)]
    cnt.append(float(HW) - tot[0] - tot[1] - tot[2] - tot[3])
    present = [cnt[s] > 0.0 for s in range(S)]
    cnt_safe = [jnp.where(present[s], cnt[s], 1.0) for s in range(S)]
    k = jnp.float32(0.0)
    for s in range(S):
        k = k + jnp.where(present[s], 1.0, 0.0)
    mu = []
    for s in range(S - 1):
        mu.append([tot[4 + s * C + c] / cnt_safe[s] for c in range(C)])
    lastc = []
    for c in range(C):
        v = tot[4 + 4 * C + c]
        for s in range(S - 1):
            v = v - tot[4 + s * C + c]
        lastc.append(v / cnt_safe[S - 1])
    mu.append(lastc)

    lab = lab_ref[...]
    sq = jnp.zeros(lab.shape, jnp.float32)
    for c in range(C):
        mc = jnp.full(lab.shape, mu[S - 1][c], jnp.float32)
        for s in range(S - 2, -1, -1):
            mc = jnp.where(lab == s, mu[s][c], mc)
        dis = mc - pred_ref[c]
        sq = sq + dis * dis
    nrm = jnp.where(sq > 0.0, jnp.sqrt(jnp.where(sq > 0.0, sq, 1.0)), 0.0)
    h = jnp.maximum(nrm - dv_ref[0, 0], 0.0)
    d = h * h
    part = jnp.float32(0.0)
    for s in range(S):
        segs = jnp.sum(jnp.where(lab == s, d, 0.0))
        part = part + jnp.where(present[s], segs / (cnt_safe[s] * k), 0.0)
    i = pl.program_id(0)
    row_ids = lax.broadcasted_iota(jnp.int32, (8, 128), 0)
    val = jnp.where(row_ids == i, part, 0.0)

    @pl.when(i == 0)
    def _():
        out_ref[...] = val

    @pl.when(i > 0)
    def _():
        out_ref[...] = out_ref[...] + val


@functools.lru_cache(maxsize=1)
def _build_tc():
    return pl.pallas_call(
        _tc_body,
        grid=(TC_STEPS,),
        in_specs=[
            pl.BlockSpec((1, 128), lambda i: (0, 0)),
            pl.BlockSpec((1, 128), lambda i: (0, 0)),
            pl.BlockSpec(
                (C, TC_BLK, W), lambda i: (0, (H_SC // TC_BLK) + i, 0)
            ),
            pl.BlockSpec((TC_BLK, W), lambda i: ((H_SC // TC_BLK) + i, 0)),
        ],
        out_specs=pl.BlockSpec((8, 128), lambda i: (0, 0)),
        out_shape=jax.ShapeDtypeStruct((8, 128), jnp.float32),
    )


@functools.lru_cache(maxsize=1)
def _build():
    mesh = _mesh()
    params = pltpu.CompilerParams(use_tc_tiling_on_sc=True)
    p1 = pl.kernel(
        _pass1_body,
        out_type=jax.ShapeDtypeStruct((NW * PBLK,), jnp.float32),
        mesh=mesh,
        compiler_params=params,
        scratch_types=[
            pltpu.VMEM((RPT, W), jnp.int32),
            pltpu.VMEM((C, RPT, W), jnp.float32),
            pltpu.VMEM((PBLK,), jnp.float32),
            pltpu.SemaphoreType.DMA,
        ],
    )
    p2 = pl.kernel(
        _pass2_body,
        out_type=jax.ShapeDtypeStruct((NW * L,), jnp.float32),
        mesh=mesh,
        compiler_params=params,
        scratch_types=[
            pltpu.VMEM((RPT2, W), jnp.int32),
            pltpu.VMEM((C, RPT2, W), jnp.float32),
            pltpu.VMEM((NW * PBLK,), jnp.float32),
            pltpu.VMEM((L,), jnp.float32),
            pltpu.VMEM((L,), jnp.float32),
            pltpu.VMEM((L,), jnp.float32),
            pltpu.SemaphoreType.DMA,
        ],
    )
    return p1, p2


def kernel(prediction, correct_label, delta_v, delta_d):
    pass1, pass2 = _build()
    tc_pass2 = _build_tc()
    pred = prediction.reshape(C, H, W)
    lab = correct_label.reshape(H, W).astype(jnp.int32)
    dv = jnp.full((L,), delta_v, jnp.float32)
    dd = jnp.full((L,), delta_d, jnp.float32)
    p1 = pass1(pred, lab, dv, dd)
    tot = jnp.sum(p1.reshape(NW, 32, L), axis=(0, 2))
    tot_pad = jnp.pad(tot, (0, 128 - 32)).reshape(1, 128)
    dv_pad = jnp.full((1, 128), delta_v, jnp.float32)
    parts = pass2(pred, lab, p1, dv, dd)
    tc_parts = tc_pass2(tot_pad, dv_pad, pred, lab)
    return jnp.sum(parts) + jnp.sum(tc_parts[:, 0])


# speedup vs baseline: 1.2139x; 1.0941x over previous
import functools

import jax
import jax.numpy as jnp
from jax import lax
from jax.experimental import pallas as pl
from jax.experimental.pallas import tpu as pltpu
from jax.experimental.pallas import tpu_sc as plsc

NC = 2
NS = 16
NW = NC * NS
L = 16
S = 5
C = 4
H = 512
W = 1024
HW = H * W
RPT = H // NW
PPT = RPT * W
VECS = PPT // L
CV = W // L
H_SC = H // 2
RPT1 = H_SC // NW
VECS1 = RPT1 * CV
RPT2 = H_SC // NW
VECS2 = RPT2 * CV
NROW = 4 + 4 * C + C
PBLK = 32 * L


def _mesh():
    return plsc.VectorSubcoreMesh(
        core_axis_name="c", subcore_axis_name="s", num_cores=NC, num_subcores=NS
    )


def _wid():
    return lax.axis_index("s") * NC + lax.axis_index("c")


def _vsqrt(x):
    xi = lax.bitcast_convert_type(x, jnp.int32)
    yi = jnp.int32(0x5F3759DF) - (xi >> 1)
    r = lax.bitcast_convert_type(yi, jnp.float32)
    x2 = 0.5 * x
    r = r * (1.5 - x2 * r * r)
    r = r * (1.5 - x2 * r * r)
    return jnp.where(x > 0.0, x * r, 0.0)


def _hsum(v):
    idx = lax.iota(jnp.int32, L)
    for sh in (8, 4, 2, 1):
        v = v + v.at[idx ^ sh].get(mode="promise_in_bounds")
    return v


def _vec(i):
    return i >> 6, pl.multiple_of((i & (CV - 1)) << 4, L)


def _pass1_body(pred_hbm, lab_hbm, dv_hbm, dd_hbm, out_hbm,
                lab_v, pred_v, part_v, sem):
    del dv_hbm, dd_hbm
    wid = _wid()
    r0 = wid * RPT1
    cps = [pltpu.async_copy(lab_hbm.at[pl.ds(r0, RPT1), :], lab_v, sem)]
    for c in range(C):
        cps.append(
            pltpu.async_copy(pred_hbm.at[c, pl.ds(r0, RPT1), :], pred_v.at[c], sem)
        )
    for cp in cps:
        cp.wait()

    zero = jnp.zeros((L,), jnp.float32)

    def body(i, acc):
        cnt, sums, tot = acc
        r, cc = _vec(i)
        lab16 = lab_v[r, pl.ds(cc, L)]
        p = [pred_v[c, r, pl.ds(cc, L)] for c in range(C)]
        cnt = list(cnt)
        sums = [list(row) for row in sums]
        tot = list(tot)
        for s in range(S - 1):
            m = lab16 == s
            cnt[s] = cnt[s] + jnp.where(m, 1.0, 0.0)
            for c in range(C):
                sums[s][c] = sums[s][c] + jnp.where(m, p[c], 0.0)
        for c in range(C):
            tot[c] = tot[c] + p[c]
        return (
            tuple(cnt),
            tuple(tuple(row) for row in sums),
            tuple(tot),
        )

    cnt0 = tuple(zero for _ in range(S - 1))
    sums0 = tuple(tuple(zero for _ in range(C)) for _ in range(S - 1))
    tot0 = tuple(zero for _ in range(C))
    cnt, sums, tot = plsc.parallel_loop(
        0, VECS1, carry=(cnt0, sums0, tot0), unroll=4
    )(body)

    for s in range(S - 1):
        part_v[pl.ds(s * L, L)] = cnt[s]
        for c in range(C):
            part_v[pl.ds((4 + s * C + c) * L, L)] = sums[s][c]
    for c in range(C):
        part_v[pl.ds((4 + 4 * C + c) * L, L)] = tot[c]
    pltpu.sync_copy(part_v, out_hbm.at[pl.ds(wid * PBLK, PBLK)])


def _pass2_body(pred_hbm, lab_hbm, p1_hbm, dv_hbm, dd_hbm, out_hbm,
                lab_v, pred_v, p1_v, dv_v, dd_v, outv, sem):
    wid = _wid()
    r0 = wid * RPT2
    cps = [pltpu.async_copy(lab_hbm.at[pl.ds(r0, RPT2), :], lab_v, sem)]
    for c in range(C):
        cps.append(
            pltpu.async_copy(pred_hbm.at[c, pl.ds(r0, RPT2), :], pred_v.at[c], sem)
        )
    cps.append(pltpu.async_copy(p1_hbm, p1_v, sem))
    cps.append(pltpu.async_copy(dv_hbm, dv_v, sem))
    cps.append(pltpu.async_copy(dd_hbm, dd_v, sem))
    for cp in cps:
        cp.wait()
    dv = dv_v[...]
    dd = dd_v[...]

    def comb(t, acc):
        return tuple(
            acc[j] + p1_v[pl.ds(t * PBLK + j * L, L)] for j in range(NROW)
        )

    tot = lax.fori_loop(
        1, NW + 1, comb, tuple(p1_v[pl.ds(j * L, L)] for j in range(NROW))
    )

    one = jnp.ones((L,), jnp.float32)
    zero = jnp.zeros((L,), jnp.float32)

    cnt = [_hsum(tot[s]) for s in range(S - 1)]
    cnt4 = jnp.full((L,), float(HW), jnp.float32)
    for s in range(S - 1):
        cnt4 = cnt4 - cnt[s]
    cnt.append(cnt4)
    present = [cnt[s] > 0.0 for s in range(S)]
    cnt_safe = [jnp.where(present[s], cnt[s], one) for s in range(S)]
    kvec = zero
    for s in range(S):
        kvec = kvec + jnp.where(present[s], one, zero)
    sums = [[_hsum(tot[4 + s * C + c]) for c in range(C)] for s in range(S - 1)]
    last = []
    for c in range(C):
        sc = _hsum(tot[4 + 4 * C + c])
        for s in range(S - 1):
            sc = sc - sums[s][c]
        last.append(sc)
    sums.append(last)
    mu = [
        [sums[s][c] / cnt_safe[s] for c in range(C)]
        for s in range(S)
    ]

    def body(i, acc):
        seg, totd = acc
        r, cc = _vec(i)
        lab16 = lab_v[r, pl.ds(cc, L)]
        p = [pred_v[c, r, pl.ds(cc, L)] for c in range(C)]
        masks = [lab16 == s for s in range(S - 1)]
        sq = zero
        for c in range(C):
            mc = mu[S - 1][c]
            for s in range(S - 2, -1, -1):
                mc = jnp.where(masks[s], mu[s][c], mc)
            dis = mc - p[c]
            sq = sq + dis * dis
        nrm = _vsqrt(sq)
        h = jnp.maximum(nrm - dv, 0.0)
        d = h * h
        seg = tuple(
            seg[s] + jnp.where(masks[s], d, zero) for s in range(S - 1)
        )
        return seg, totd + d

    seg, totd = plsc.parallel_loop(
        0, VECS2, carry=(tuple(zero for _ in range(S - 1)), zero), unroll=4
    )(body)
    seg = list(seg)
    seg4 = totd
    for s in range(S - 1):
        seg4 = seg4 - seg[s]
    seg.append(seg4)

    part = zero
    for s in range(S):
        part = part + jnp.where(
            present[s], seg[s] / (cnt_safe[s] * kvec), zero
        )

    acc = zero
    for a in range(S):
        for b in range(a + 1, S):
            sq2 = zero
            for c in range(C):
                df = mu[a][c] - mu[b][c]
                sq2 = sq2 + df * df
            dist = _vsqrt(sq2)
            hg = jnp.maximum(dd - dist, 0.0)
            pm = jnp.where(present[a], one, zero) * jnp.where(
                present[b], one, zero
            )
            acc = acc + 2.0 * pm * hg * hg
    l_dist = acc / (kvec * (kvec - one))
    part = part + l_dist * (1.0 / (NW * L))

    outv[...] = part
    pltpu.sync_copy(outv, out_hbm.at[pl.ds(wid * L, L)])


TC_ROWS = H - H_SC
TC_STEPS = 4
TC_BLK = TC_ROWS // TC_STEPS


def _tc_body(tot_ref, dv_ref, pred_ref, lab_ref, out_ref):
    tot = [tot_ref[0, j] for j in range(NROW)]
    cnt = [tot[s] for s in range(S - 1)]
    cnt.append(float(HW) - tot[0] - tot[1] - tot[2] - tot[3])
    present = [cnt[s] > 0.0 for s in range(S)]
    cnt_safe = [jnp.where(present[s], cnt[s], 1.0) for s in range(S)]
    k = jnp.float32(0.0)
    for s in range(S):
        k = k + jnp.where(present[s], 1.0, 0.0)
    mu = []
    for s in range(S - 1):
        mu.append([tot[4 + s * C + c] / cnt_safe[s] for c in range(C)])
    lastc = []
    for c in range(C):
        v = tot[4 + 4 * C + c]
        for s in range(S - 1):
            v = v - tot[4 + s * C + c]
        lastc.append(v / cnt_safe[S - 1])
    mu.append(lastc)

    lab = lab_ref[...]
    sq = jnp.zeros(lab.shape, jnp.float32)
    for c in range(C):
        mc = jnp.full(lab.shape, mu[S - 1][c], jnp.float32)
        for s in range(S - 2, -1, -1):
            mc = jnp.where(lab == s, mu[s][c], mc)
        dis = mc - pred_ref[c]
        sq = sq + dis * dis
    nrm = jnp.where(sq > 0.0, jnp.sqrt(jnp.where(sq > 0.0, sq, 1.0)), 0.0)
    h = jnp.maximum(nrm - dv_ref[0, 0], 0.0)
    d = h * h
    part = jnp.float32(0.0)
    for s in range(S):
        segs = jnp.sum(jnp.where(lab == s, d, 0.0))
        part = part + jnp.where(present[s], segs / (cnt_safe[s] * k), 0.0)
    i = pl.program_id(0)
    row_ids = lax.broadcasted_iota(jnp.int32, (8, 128), 0)
    val = jnp.where(row_ids == i, part, 0.0)

    @pl.when(i == 0)
    def _():
        out_ref[...] = val

    @pl.when(i > 0)
    def _():
        out_ref[...] = out_ref[...] + val


def _tc1_body(pred_ref, lab_ref, out_ref):
    lab = lab_ref[...]
    vals = []
    for s in range(S - 1):
        vals.append(jnp.sum(jnp.where(lab == s, 1.0, 0.0)))
    for s in range(S - 1):
        m = lab == s
        for c in range(C):
            vals.append(jnp.sum(jnp.where(m, pred_ref[c], 0.0)))
    for c in range(C):
        vals.append(jnp.sum(pred_ref[c]))
    i = pl.program_id(0)
    row_ids = lax.broadcasted_iota(jnp.int32, (8, 128), 0)
    col_ids = lax.broadcasted_iota(jnp.int32, (8, 128), 1)
    blk = jnp.zeros((8, 128), jnp.float32)
    for j, v in enumerate(vals):
        blk = jnp.where((row_ids == 0) & (col_ids == j), v, blk)

    @pl.when(i == 0)
    def _():
        out_ref[...] = blk

    @pl.when(i > 0)
    def _():
        out_ref[...] = out_ref[...] + blk


@functools.lru_cache(maxsize=1)
def _build_tc1():
    return pl.pallas_call(
        _tc1_body,
        grid=(TC_STEPS,),
        in_specs=[
            pl.BlockSpec(
                (C, TC_BLK, W), lambda i: (0, (H_SC // TC_BLK) + i, 0)
            ),
            pl.BlockSpec((TC_BLK, W), lambda i: ((H_SC // TC_BLK) + i, 0)),
        ],
        out_specs=pl.BlockSpec((8, 128), lambda i: (0, 0)),
        out_shape=jax.ShapeDtypeStruct((8, 128), jnp.float32),
    )


@functools.lru_cache(maxsize=1)
def _build_tc():
    return pl.pallas_call(
        _tc_body,
        grid=(TC_STEPS,),
        in_specs=[
            pl.BlockSpec((1, 128), lambda i: (0, 0)),
            pl.BlockSpec((1, 128), lambda i: (0, 0)),
            pl.BlockSpec(
                (C, TC_BLK, W), lambda i: (0, (H_SC // TC_BLK) + i, 0)
            ),
            pl.BlockSpec((TC_BLK, W), lambda i: ((H_SC // TC_BLK) + i, 0)),
        ],
        out_specs=pl.BlockSpec((8, 128), lambda i: (0, 0)),
        out_shape=jax.ShapeDtypeStruct((8, 128), jnp.float32),
    )


@functools.lru_cache(maxsize=1)
def _build():
    mesh = _mesh()
    params = pltpu.CompilerParams(use_tc_tiling_on_sc=True)
    p1 = pl.kernel(
        _pass1_body,
        out_type=jax.ShapeDtypeStruct((NW * PBLK,), jnp.float32),
        mesh=mesh,
        compiler_params=params,
        scratch_types=[
            pltpu.VMEM((RPT1, W), jnp.int32),
            pltpu.VMEM((C, RPT1, W), jnp.float32),
            pltpu.VMEM((PBLK,), jnp.float32),
            pltpu.SemaphoreType.DMA,
        ],
    )
    p2 = pl.kernel(
        _pass2_body,
        out_type=jax.ShapeDtypeStruct((NW * L,), jnp.float32),
        mesh=mesh,
        compiler_params=params,
        scratch_types=[
            pltpu.VMEM((RPT2, W), jnp.int32),
            pltpu.VMEM((C, RPT2, W), jnp.float32),
            pltpu.VMEM(((NW + 1) * PBLK,), jnp.float32),
            pltpu.VMEM((L,), jnp.float32),
            pltpu.VMEM((L,), jnp.float32),
            pltpu.VMEM((L,), jnp.float32),
            pltpu.SemaphoreType.DMA,
        ],
    )
    return p1, p2


def kernel(prediction, correct_label, delta_v, delta_d):
    pass1, pass2 = _build()
    tc_pass2 = _build_tc()
    pred = prediction.reshape(C, H, W)
    lab = correct_label.reshape(H, W).astype(jnp.int32)
    dv = jnp.full((L,), delta_v, jnp.float32)
    dd = jnp.full((L,), delta_d, jnp.float32)
    p1_sc = pass1(pred, lab, dv, dd)
    tc1 = _build_tc1()(pred, lab)
    tc1_blk = (
        jnp.zeros((PBLK,), jnp.float32)
        .at[jnp.arange(NROW) * L]
        .set(tc1[0, :NROW])
    )
    p1 = jnp.concatenate([p1_sc, tc1_blk])
    tot = jnp.sum(p1.reshape(NW + 1, 32, L), axis=(0, 2))
    tot_pad = jnp.pad(tot, (0, 128 - 32)).reshape(1, 128)
    dv_pad = jnp.full((1, 128), delta_v, jnp.float32)
    parts = pass2(pred, lab, p1, dv, dd)
    tc_parts = tc_pass2(tot_pad, dv_pad, pred, lab)
    return jnp.sum(parts) + jnp.sum(tc_parts[:, 0])
